# baseline (speedup 1.0000x reference)
import functools

import jax
import jax.numpy as jnp
from jax.experimental import pallas as pl
from jax.experimental.pallas import tpu as pltpu

_Q = 32
_K = 100000
_D = 2048
_KB = 1000


def _body(q_ref, k_ref, idx_ref, score_ref, *, kb, nblk, total_k):
    j = pl.program_id(0)
    q = q_ref[...]
    k = k_ref[...]

    scores = jax.lax.dot_general(
        q, k, (((1,), (1,)), ((), ())),
        preferred_element_type=jnp.float32,
        precision=jax.lax.Precision.DEFAULT,
    )
    k_norm = jnp.sqrt(jnp.sum(k * k, axis=1))
    sim = scores / k_norm[None, :]

    local_max = jnp.max(sim, axis=1, keepdims=True)
    lanes = jax.lax.broadcasted_iota(jnp.int32, sim.shape, 1)
    local_idx = jnp.min(
        jnp.where(sim == local_max, lanes, jnp.int32(total_k)),
        axis=1, keepdims=True,
    ) + j * kb

    @pl.when(j == 0)
    def _init():
        score_ref[...] = local_max
        idx_ref[...] = local_idx

    @pl.when(j > 0)
    def _merge():
        prev = score_ref[...]
        better = local_max > prev
        score_ref[...] = jnp.where(better, local_max, prev)
        idx_ref[...] = jnp.where(better, local_idx, idx_ref[...])

    @pl.when(j == nblk - 1)
    def _finalize():
        q_norm = jnp.sqrt(jnp.sum(q * q, axis=1, keepdims=True))
        score_ref[...] = score_ref[...] / q_norm


@functools.partial(jax.jit, static_argnames=())
def kernel(queries, keys):
    q, d = queries.shape
    k, _ = keys.shape
    nblk = k // _KB
    assert nblk * _KB == k

    body = functools.partial(_body, kb=_KB, nblk=nblk, total_k=k)
    idx2, score2 = pl.pallas_call(
        body,
        grid=(nblk,),
        in_specs=[
            pl.BlockSpec((q, d), lambda j: (0, 0)),
            pl.BlockSpec((_KB, d), lambda j: (j, 0)),
        ],
        out_specs=[
            pl.BlockSpec((q, 1), lambda j: (0, 0)),
            pl.BlockSpec((q, 1), lambda j: (0, 0)),
        ],
        out_shape=[
            jax.ShapeDtypeStruct((q, 1), jnp.int32),
            jax.ShapeDtypeStruct((q, 1), jnp.float32),
        ],
        compiler_params=pltpu.CompilerParams(
            dimension_semantics=("arbitrary",),
        ),
    )(queries, keys)
    return idx2.reshape(q), score2.reshape(q)


# speedup vs baseline: 1.0932x; 1.0932x over previous
import functools

import jax
import jax.numpy as jnp
from jax.experimental import pallas as pl
from jax.experimental.pallas import tpu as pltpu

_Q = 32
_K = 100000
_D = 2048
_KB = 2000


def _body(q_ref, k_ref, idx_ref, score_ref, *, kb, nblk, total_k):
    j = pl.program_id(0)
    q = q_ref[...]
    k = k_ref[...]

    scores = jax.lax.dot_general(
        q, k, (((1,), (1,)), ((), ())),
        preferred_element_type=jnp.float32,
        precision=jax.lax.Precision.DEFAULT,
    )
    k_norm = jnp.sqrt(jnp.sum(k * k, axis=1))
    sim = scores / k_norm[None, :]

    local_max = jnp.max(sim, axis=1, keepdims=True)
    lanes = jax.lax.broadcasted_iota(jnp.int32, sim.shape, 1)
    local_idx = jnp.min(
        jnp.where(sim == local_max, lanes, jnp.int32(total_k)),
        axis=1, keepdims=True,
    ) + j * kb

    @pl.when(j == 0)
    def _init():
        score_ref[...] = local_max
        idx_ref[...] = local_idx

    @pl.when(j > 0)
    def _merge():
        prev = score_ref[...]
        better = local_max > prev
        score_ref[...] = jnp.where(better, local_max, prev)
        idx_ref[...] = jnp.where(better, local_idx, idx_ref[...])

    @pl.when(j == nblk - 1)
    def _finalize():
        q_norm = jnp.sqrt(jnp.sum(q * q, axis=1, keepdims=True))
        score_ref[...] = score_ref[...] / q_norm


@functools.partial(jax.jit, static_argnames=())
def kernel(queries, keys):
    q, d = queries.shape
    k, _ = keys.shape
    nblk = k // _KB
    assert nblk * _KB == k

    body = functools.partial(_body, kb=_KB, nblk=nblk, total_k=k)
    idx2, score2 = pl.pallas_call(
        body,
        grid=(nblk,),
        in_specs=[
            pl.BlockSpec((q, d), lambda j: (0, 0)),
            pl.BlockSpec((_KB, d), lambda j: (j, 0)),
        ],
        out_specs=[
            pl.BlockSpec((q, 1), lambda j: (0, 0)),
            pl.BlockSpec((q, 1), lambda j: (0, 0)),
        ],
        out_shape=[
            jax.ShapeDtypeStruct((q, 1), jnp.int32),
            jax.ShapeDtypeStruct((q, 1), jnp.float32),
        ],
        compiler_params=pltpu.CompilerParams(
            dimension_semantics=("arbitrary",),
        ),
    )(queries, keys)
    return idx2.reshape(q), score2.reshape(q)
